# Initial kernel scaffold; baseline (speedup 1.0000x reference)
import jax
import jax.numpy as jnp
from jax.experimental import pallas as pl


def kernel(ip_bits, port_indices, tech_indices, W_ip, b_ip, port_table, tech_table):
    raise NotImplementedError("write your pallas kernel here")


# speedup vs baseline: 4.0081x; 4.0081x over previous
import functools

import jax
import jax.numpy as jnp
from jax import lax
from jax.experimental import pallas as pl
from jax.experimental.pallas import tpu as pltpu
from jax.experimental.pallas import tpu_sc as plsc

_N_IP = 65536
_N_PORT = 524288
_N_TECH = 131072
_D = 64

_NC = 2
_NS = 16
_NW = _NC * _NS

_C = 512


def _sc_gather_body(port_table, port_idx, tech_table, tech_idx,
                    port_out, tech_out, idx_v, rows_v, sem):
    wid = lax.axis_index("s") * _NC + lax.axis_index("c")

    def run(table, idx_hbm, out_hbm, total):
        b_per_w = total // _NW
        nchunks = b_per_w // _C
        base = wid * b_per_w

        @pl.loop(0, nchunks)
        def _(i):
            off = pl.multiple_of(base + i * _C, _C)
            pltpu.sync_copy(idx_hbm.at[pl.ds(off, _C)], idx_v)
            pltpu.async_copy(table.at[idx_v], rows_v, sem).wait()
            pltpu.sync_copy(rows_v, out_hbm.at[pl.ds(off, _C)])

    run(port_table, port_idx, port_out, _N_PORT)
    run(tech_table, tech_idx, tech_out, _N_TECH)


@jax.jit
def _sc_gathers(port_table, port_idx, tech_table, tech_idx):
    mesh = plsc.VectorSubcoreMesh(core_axis_name="c", subcore_axis_name="s")
    return pl.kernel(
        _sc_gather_body,
        out_type=(
            jax.ShapeDtypeStruct((_N_PORT, _D), jnp.float32),
            jax.ShapeDtypeStruct((_N_TECH, _D), jnp.float32),
        ),
        mesh=mesh,
        scratch_types=[
            pltpu.VMEM((_C,), jnp.int32),
            pltpu.VMEM((_C, _D), jnp.float32),
            pltpu.SemaphoreType.DMA,
        ],
        compiler_params=pltpu.CompilerParams(use_tc_tiling_on_sc=False),
    )(port_table, port_idx, tech_table, tech_idx)


def _ip_body(x_ref, wt_ref, b_ref, o_ref):
    o_ref[...] = (
        jnp.dot(x_ref[...], wt_ref[...], preferred_element_type=jnp.float32)
        + b_ref[...]
    )


_BM = 8192


@jax.jit
def _ip_linear(ip_bits, W_ip_t, b_ip2d):
    return pl.pallas_call(
        _ip_body,
        grid=(_N_IP // _BM,),
        in_specs=[
            pl.BlockSpec((_BM, 32), lambda i: (i, 0)),
            pl.BlockSpec((32, _D), lambda i: (0, 0)),
            pl.BlockSpec((1, _D), lambda i: (0, 0)),
        ],
        out_specs=pl.BlockSpec((_BM, _D), lambda i: (i, 0)),
        out_shape=jax.ShapeDtypeStruct((_N_IP, _D), jnp.float32),
    )(ip_bits, W_ip_t, b_ip2d)


def kernel(ip_bits, port_indices, tech_indices, W_ip, b_ip, port_table, tech_table):
    port_x, tech_x = _sc_gathers(
        port_table, port_indices, tech_table, tech_indices
    )
    ip_x = _ip_linear(ip_bits, W_ip.T, b_ip.reshape(1, _D))
    return ip_x, port_x, tech_x


# speedup vs baseline: 4.2436x; 1.0587x over previous
import functools

import jax
import jax.numpy as jnp
from jax import lax
from jax.experimental import pallas as pl
from jax.experimental.pallas import tpu as pltpu
from jax.experimental.pallas import tpu_sc as plsc

_N_IP = 65536
_N_PORT = 524288
_N_TECH = 131072
_D = 64

_NC = 2
_NS = 16
_NW = _NC * _NS

_C = 512
_NBUF = 3
_IDX_MAX = _N_PORT // _NW


def _sc_gather_body(port_table, port_idx, tech_table, tech_idx,
                    port_out, tech_out, idx_all, rows_v, gsem, ssem):
    wid = lax.axis_index("s") * _NC + lax.axis_index("c")

    def run(table, idx_hbm, out_hbm, total):
        b_per_w = total // _NW
        nchunks = b_per_w // _C
        base = wid * b_per_w

        pltpu.sync_copy(idx_hbm.at[pl.ds(base, b_per_w)], idx_all.at[pl.ds(0, b_per_w)])

        def issue(c, b):
            idx_slice = idx_all.at[pl.ds(c * _C, _C)]
            pltpu.async_copy(table.at[idx_slice], rows_v.at[b], gsem.at[b])

        def complete(c, b):
            idx_slice = idx_all.at[pl.ds(c * _C, _C)]
            pltpu.make_async_copy(table.at[idx_slice], rows_v.at[b], gsem.at[b]).wait()
            pltpu.async_copy(rows_v.at[b], out_hbm.at[pl.ds(base + c * _C, _C)], ssem.at[b])

        def wait_store(c, b):
            pltpu.make_async_copy(
                rows_v.at[b], out_hbm.at[pl.ds(base + c * _C, _C)], ssem.at[b]
            ).wait()

        for c in range(_NBUF):
            issue(c, c)
        complete(0, 0)

        @pl.loop(_NBUF, nchunks)
        def _(c):
            b = lax.rem(c, _NBUF)
            bp = lax.rem(c - 2, _NBUF)
            wait_store(c - _NBUF, b)
            issue(c, b)
            complete(c - 2, bp)

        complete(nchunks - 2, (nchunks - 2) % _NBUF)
        complete(nchunks - 1, (nchunks - 1) % _NBUF)
        for c in range(nchunks - _NBUF, nchunks):
            wait_store(c, c % _NBUF)

    run(port_table, port_idx, port_out, _N_PORT)
    run(tech_table, tech_idx, tech_out, _N_TECH)


@jax.jit
def _sc_gathers(port_table, port_idx, tech_table, tech_idx):
    mesh = plsc.VectorSubcoreMesh(core_axis_name="c", subcore_axis_name="s")
    return pl.kernel(
        _sc_gather_body,
        out_type=(
            jax.ShapeDtypeStruct((_N_PORT, _D), jnp.float32),
            jax.ShapeDtypeStruct((_N_TECH, _D), jnp.float32),
        ),
        mesh=mesh,
        scratch_types=[
            pltpu.VMEM((_IDX_MAX,), jnp.int32),
            pltpu.VMEM((_NBUF, _C, _D), jnp.float32),
            pltpu.SemaphoreType.DMA((_NBUF,)),
            pltpu.SemaphoreType.DMA((_NBUF,)),
        ],
        compiler_params=pltpu.CompilerParams(use_tc_tiling_on_sc=False),
    )(port_table, port_idx, tech_table, tech_idx)


def _ip_body(x_ref, wt_ref, b_ref, o_ref):
    o_ref[...] = (
        jnp.dot(x_ref[...], wt_ref[...], preferred_element_type=jnp.float32)
        + b_ref[...]
    )


_BM = 8192


@jax.jit
def _ip_linear(ip_bits, W_ip_t, b_ip2d):
    return pl.pallas_call(
        _ip_body,
        grid=(_N_IP // _BM,),
        in_specs=[
            pl.BlockSpec((_BM, 32), lambda i: (i, 0)),
            pl.BlockSpec((32, _D), lambda i: (0, 0)),
            pl.BlockSpec((1, _D), lambda i: (0, 0)),
        ],
        out_specs=pl.BlockSpec((_BM, _D), lambda i: (i, 0)),
        out_shape=jax.ShapeDtypeStruct((_N_IP, _D), jnp.float32),
    )(ip_bits, W_ip_t, b_ip2d)


def kernel(ip_bits, port_indices, tech_indices, W_ip, b_ip, port_table, tech_table):
    port_x, tech_x = _sc_gathers(
        port_table, port_indices, tech_table, tech_indices
    )
    ip_x = _ip_linear(ip_bits, W_ip.T, b_ip.reshape(1, _D))
    return ip_x, port_x, tech_x
